# Initial kernel scaffold; baseline (speedup 1.0000x reference)
import jax
import jax.numpy as jnp
from jax.experimental import pallas as pl


def kernel(x, adj, weight, bias):
    raise NotImplementedError("write your pallas kernel here")


# speedup vs baseline: 1.0158x; 1.0158x over previous
import functools

import jax
import jax.numpy as jnp
from jax.experimental import pallas as pl

B, N, F_IN, F_OUT = 2, 4096, 128, 128
BLK = 512


def _support_kernel(x_ref, w_ref, s_ref):
    s = jnp.dot(x_ref[0], w_ref[...], preferred_element_type=jnp.float32)
    s_ref[0] = s.astype(jnp.bfloat16)


def _agg_kernel(adj_ref, s_ref, b_ref, o_ref):
    a = adj_ref[...].astype(jnp.bfloat16)
    bias = b_ref[0]
    o_ref[0] = jnp.dot(a, s_ref[0], preferred_element_type=jnp.float32) + bias
    o_ref[1] = jnp.dot(a, s_ref[1], preferred_element_type=jnp.float32) + bias


@functools.partial(jax.jit, static_argnames=())
def kernel(x, adj, weight, bias):
    support = pl.pallas_call(
        _support_kernel,
        grid=(B,),
        in_specs=[
            pl.BlockSpec((1, N, F_IN), lambda b: (b, 0, 0)),
            pl.BlockSpec((F_IN, F_OUT), lambda b: (0, 0)),
        ],
        out_specs=pl.BlockSpec((1, N, F_OUT), lambda b: (b, 0, 0)),
        out_shape=jax.ShapeDtypeStruct((B, N, F_OUT), jnp.bfloat16),
    )(x, weight)

    out = pl.pallas_call(
        _agg_kernel,
        grid=(N // BLK,),
        in_specs=[
            pl.BlockSpec((BLK, N), lambda i: (i, 0)),
            pl.BlockSpec((B, N, F_OUT), lambda i: (0, 0, 0)),
            pl.BlockSpec((1, F_OUT), lambda i: (0, 0)),
        ],
        out_specs=pl.BlockSpec((B, BLK, F_OUT), lambda i: (0, i, 0)),
        out_shape=jax.ShapeDtypeStruct((B, N, F_OUT), jnp.float32),
    )(adj, support, bias.reshape(1, F_OUT))
    return out


# speedup vs baseline: 1.2602x; 1.2406x over previous
import jax
import jax.numpy as jnp
from jax.experimental import pallas as pl
from jax.experimental.pallas import tpu as pltpu

B, N, F_IN, F_OUT = 2, 4096, 128, 128
BLK = 512


def _gcn_kernel(adj_ref, x_ref, w_ref, b_ref, o_ref, s_ref):
    i = pl.program_id(0)

    @pl.when(i == 0)
    def _():
        w = w_ref[...]
        s0 = jnp.dot(x_ref[0], w, preferred_element_type=jnp.float32)
        s1 = jnp.dot(x_ref[1], w, preferred_element_type=jnp.float32)
        s_ref[:, :F_OUT] = s0.astype(jnp.bfloat16)
        s_ref[:, F_OUT:] = s1.astype(jnp.bfloat16)

    a = adj_ref[...].astype(jnp.bfloat16)
    r = jnp.dot(a, s_ref[...], preferred_element_type=jnp.float32)
    bias = b_ref[0]
    o_ref[0] = r[:, :F_OUT] + bias
    o_ref[1] = r[:, F_OUT:] + bias


def kernel(x, adj, weight, bias):
    return pl.pallas_call(
        _gcn_kernel,
        grid=(N // BLK,),
        in_specs=[
            pl.BlockSpec((BLK, N), lambda i: (i, 0)),
            pl.BlockSpec((B, N, F_IN), lambda i: (0, 0, 0)),
            pl.BlockSpec((F_IN, F_OUT), lambda i: (0, 0)),
            pl.BlockSpec((1, F_OUT), lambda i: (0, 0)),
        ],
        out_specs=pl.BlockSpec((B, BLK, F_OUT), lambda i: (0, i, 0)),
        out_shape=jax.ShapeDtypeStruct((B, N, F_OUT), jnp.float32),
        scratch_shapes=[pltpu.VMEM((N, B * F_OUT), jnp.bfloat16)],
    )(adj, x, weight, bias.reshape(1, F_OUT))
